# Initial kernel scaffold; baseline (speedup 1.0000x reference)
import jax
import jax.numpy as jnp
from jax.experimental import pallas as pl


def kernel(x, edge_index, edge_attr, batch, atom_emb, lin_W, lin_b, root_emb, bond_e0, bond_e1, bond_e2, bn_w, bn_b):
    raise NotImplementedError("write your pallas kernel here")


# speedup vs baseline: 1.5357x; 1.5357x over previous
import functools

import jax
import jax.numpy as jnp
from jax import lax
from jax.experimental import pallas as pl
from jax.experimental.pallas import tpu as pltpu
from jax.experimental.pallas import tpu_sc as plsc

N = 10000
E = 320000
D = 128
L = 3
G = 64

NC = 2
NS = 16
NW = NC * NS

NP = 10240
EP = ((E + NW * 128 - 1) // (NW * 128)) * (NW * 128)
ER = EP // 128
CPW = ER // NW
HIST_PT = ER // NS
NPT = NP // NS
NPW = NP // NW

_MESH = plsc.VectorSubcoreMesh(core_axis_name="c", subcore_axis_name="s",
                               num_cores=NC, num_subcores=NS)


@functools.partial(
    pl.kernel,
    out_type=jax.ShapeDtypeStruct((NC * NP,), jnp.float32),
    mesh=_MESH,
    scratch_types=[
        pltpu.VMEM_SHARED((NP, 16), jnp.float32),
        pltpu.VMEM((128, 16), jnp.float32),
        pltpu.VMEM((128,), jnp.int32),
        pltpu.VMEM((NPT, 16), jnp.float32),
        pltpu.VMEM((NPT,), jnp.float32),
    ],
    compiler_params=pltpu.CompilerParams(needs_layout_passes=False),
)
def _hist_kernel(row2, hist_out, hist_sh, ones_v, ridx, dbuf, disv):
    core = lax.axis_index("c")
    sid = lax.axis_index("s")

    one16 = jnp.full((16,), 1.0, jnp.float32)
    zero16 = jnp.zeros((16,), jnp.float32)
    for j in range(128):
        ones_v[j, :] = one16
    for j in range(NPT):
        dbuf[j, :] = zero16
    pltpu.sync_copy(dbuf, hist_sh.at[pl.ds(sid * NPT, NPT)])
    plsc.subcore_barrier()

    def _hist(c, _):
        rr = ((core * NS + sid) * CPW + c) * 128
        pltpu.sync_copy(row2.at[pl.ds(rr, 128)], ridx)
        pltpu.sync_copy(ones_v, hist_sh.at[ridx], add=True)
        return 0
    lax.fori_loop(0, CPW, _hist, 0)
    plsc.subcore_barrier()

    pltpu.sync_copy(hist_sh.at[pl.ds(sid * NPT, NPT)], dbuf)
    zidx = jnp.zeros((16,), jnp.int32)
    for g in range(NPT // 16):
        rowi = jnp.int32(g * 16) + lax.iota(jnp.int32, 16)
        disv[pl.ds(g * 16, 16)] = plsc.load_gather(dbuf, [rowi, zidx])

    pltpu.sync_copy(disv, hist_out.at[pl.ds(core * NP + sid * NPT, NPT)])


@functools.partial(
    pl.kernel,
    out_type=(
        jax.ShapeDtypeStruct((EP,), jnp.float32),
        jax.ShapeDtypeStruct((EP,), jnp.int32),
    ),
    mesh=_MESH,
    scratch_types=[
        pltpu.VMEM((128,), jnp.int32),
        pltpu.VMEM((128,), jnp.int32),
        pltpu.VMEM((128,), jnp.int32),
        pltpu.VMEM((128,), jnp.int32),
        pltpu.VMEM((128,), jnp.int32),
        pltpu.VMEM((128,), jnp.float32),
        pltpu.VMEM((128,), jnp.int32),
        pltpu.VMEM((NP,), jnp.float32),
    ],
    compiler_params=pltpu.CompilerParams(needs_layout_passes=False),
)
def _norm_kernel(row2, col2, ea0, ea1, ea2, dis, norm_out, cid_out,
                 ridx, cdx, av0, av1, av2, nmv, cidv, dis_tab):
    core = lax.axis_index("c")
    sid = lax.axis_index("s")
    w = core * NS + sid

    pltpu.sync_copy(dis, dis_tab)

    def _edge(c, _):
        rr = (w * CPW + c) * 128
        sl128 = pl.ds(rr, 128)
        pltpu.sync_copy(row2.at[sl128], ridx)
        pltpu.sync_copy(col2.at[sl128], cdx)
        pltpu.sync_copy(ea0.at[sl128], av0)
        pltpu.sync_copy(ea1.at[sl128], av1)
        pltpu.sync_copy(ea2.at[sl128], av2)
        for g in range(8):
            sl = pl.ds(g * 16, 16)
            dr = plsc.load_gather(dis_tab, [ridx[sl]])
            dc = plsc.load_gather(dis_tab, [cdx[sl]])
            nmv[sl] = dr * dc
            cidv[sl] = av0[sl] * 12 + av1[sl] * 2 + av2[sl]
        pltpu.sync_copy(nmv, norm_out.at[sl128])
        pltpu.sync_copy(cidv, cid_out.at[sl128])
        return 0
    lax.fori_loop(0, CPW, _edge, 0)


@functools.partial(
    pl.kernel,
    out_type=jax.ShapeDtypeStruct((NP, D), jnp.float32),
    mesh=_MESH,
    scratch_types=[
        pltpu.VMEM((64,), jnp.int32),
        pltpu.VMEM((64,), jnp.int32),
        pltpu.VMEM((64, D), jnp.float32),
        pltpu.VMEM((64, D), jnp.float32),
        pltpu.SemaphoreType.DMA,
    ],
    compiler_params=pltpu.CompilerParams(needs_layout_passes=False),
)
def _atom_kernel(xT, aef, h0, xv, idxv, acc, gbuf, sem):
    core = lax.axis_index("c")
    sid = lax.axis_index("s")
    w = core * NS + sid

    def _chunk(c, _):
        nb = w * NPW + c * 64
        for i in range(9):
            pltpu.sync_copy(xT.at[pl.ds(i * NP + nb, 64)], xv)
            for g in range(4):
                sl = pl.ds(g * 16, 16)
                idxv[sl] = xv[sl] + jnp.int32(119 * i)
            if i == 0:
                pltpu.async_copy(aef.at[idxv], acc, sem).wait()
            else:
                pltpu.async_copy(aef.at[idxv], gbuf, sem).wait()

                def _accum(j, _):
                    for g2 in range(8):
                        sl2 = pl.ds(g2 * 16, 16)
                        acc[j, sl2] = acc[j, sl2] + gbuf[j, sl2]
                    return 0
                lax.fori_loop(0, 64, _accum, 0)
        pltpu.sync_copy(acc, h0.at[pl.ds(nb, 64)])
        return 0
    lax.fori_loop(0, NPW // 64, _chunk, 0)


@functools.partial(
    pl.kernel,
    out_type=jax.ShapeDtypeStruct((NC, NP, D), jnp.float32),
    mesh=_MESH,
    scratch_types=[
        pltpu.VMEM_SHARED((NP, D), jnp.float32),
        pltpu.VMEM((64, D), jnp.float32),
        pltpu.VMEM((128,), jnp.int32),
        pltpu.VMEM((128,), jnp.int32),
        pltpu.VMEM((128,), jnp.int32),
        pltpu.VMEM((128,), jnp.float32),
        pltpu.VMEM((128, D), jnp.float32),
        pltpu.SemaphoreType.DMA,
    ],
    compiler_params=pltpu.CompilerParams(needs_layout_passes=False),
)
def _edge_kernel(h, row2, col2, cid2, norm2, ct, agg2,
                 agg_sh, ct_v, ridx, cdx, cidv, nmv, hrow, sem):
    core = lax.axis_index("c")
    sid = lax.axis_index("s")
    w = core * NS + sid

    zero16 = jnp.zeros((16,), jnp.float32)
    for j in range(128):
        for g in range(8):
            hrow[j, pl.ds(g * 16, 16)] = zero16
    for t in range(NPT // 128):
        pltpu.sync_copy(hrow, agg_sh.at[pl.ds(sid * NPT + t * 128, 128)])
    pltpu.sync_copy(ct, ct_v)
    plsc.subcore_barrier()

    def _chunk(c, _):
        rr = (w * CPW + c) * 128
        sl128 = pl.ds(rr, 128)
        pltpu.sync_copy(row2.at[sl128], ridx)
        pltpu.sync_copy(cid2.at[sl128], cidv)
        pltpu.sync_copy(norm2.at[sl128], nmv)
        pltpu.sync_copy(col2.at[sl128], cdx)
        pltpu.async_copy(h.at[ridx], hrow, sem).wait()

        for gg in range(8):
            sl = pl.ds(gg * 16, 16)
            ne16 = nmv[sl]
            ce16 = cidv[sl]
            eidx = jnp.int32(gg * 16) + lax.iota(jnp.int32, 16)

            def _col(cc, _, ne16=ne16, ce16=ce16, eidx=eidx):
                cvec = jnp.full((16,), cc, dtype=jnp.int32)
                hv = plsc.load_gather(hrow, [eidx, cvec])
                ev = plsc.load_gather(ct_v, [ce16, cvec])
                m = jnp.maximum(hv + ev, 0.0) * ne16
                plsc.store_scatter(hrow, [eidx, cvec], m)
                return 0
            lax.fori_loop(0, 128, _col, 0, unroll=8)
        pltpu.sync_copy(hrow, agg_sh.at[cdx], add=True)
        return 0
    lax.fori_loop(0, CPW, _chunk, 0)
    plsc.subcore_barrier()

    pltpu.sync_copy(agg_sh.at[pl.ds(sid * NPT, NPT)],
                    agg2.at[core].at[pl.ds(sid * NPT, NPT)])


_BLK = 1024


def _dis_body(h0_ref, h1_ref, o_ref):
    o_ref[...] = lax.rsqrt(h0_ref[...] + h1_ref[...] + 1.0)


def _tc_dis(hist0, hist1):
    return pl.pallas_call(
        _dis_body,
        out_shape=jax.ShapeDtypeStruct((NP, 1), jnp.float32),
    )(hist0, hist1)


def _mm_body(x_ref, w_ref, b_ref, o_ref):
    o_ref[...] = jnp.dot(x_ref[...], w_ref[...],
                         preferred_element_type=jnp.float32) + b_ref[...]


def _tc_matmul(hx, wT, b):
    return pl.pallas_call(
        _mm_body,
        grid=(NP // _BLK,),
        in_specs=[
            pl.BlockSpec((_BLK, D), lambda i: (i, 0)),
            pl.BlockSpec((D, D), lambda i: (0, 0)),
            pl.BlockSpec((1, D), lambda i: (0, 0)),
        ],
        out_specs=pl.BlockSpec((_BLK, D), lambda i: (i, 0)),
        out_shape=jax.ShapeDtypeStruct((NP, D), jnp.float32),
    )(hx, wT, b)


def _ewmm_body(a0_ref, a1_ref, hp_ref, dv_ref, root_ref, s_ref, bb_ref,
               w_ref, wb_ref, o_ref):
    dd = dv_ref[...]
    hm = (a0_ref[...] + a1_ref[...]
          + jnp.maximum(hp_ref[...] + root_ref[...], 0.0) * (dd * dd))
    hm = hm * s_ref[...] + bb_ref[...]
    hm = jnp.maximum(hm, 0.0)
    o_ref[...] = jnp.dot(hm, w_ref[...],
                         preferred_element_type=jnp.float32) + wb_ref[...]


def _tc_ewmm(a0, a1, hp, dv, root, s, bb, wT, wb):
    return pl.pallas_call(
        _ewmm_body,
        grid=(NP // _BLK,),
        in_specs=[
            pl.BlockSpec((_BLK, D), lambda i: (i, 0)),
            pl.BlockSpec((_BLK, D), lambda i: (i, 0)),
            pl.BlockSpec((_BLK, D), lambda i: (i, 0)),
            pl.BlockSpec((_BLK, 1), lambda i: (i, 0)),
            pl.BlockSpec((1, D), lambda i: (0, 0)),
            pl.BlockSpec((1, D), lambda i: (0, 0)),
            pl.BlockSpec((1, D), lambda i: (0, 0)),
            pl.BlockSpec((D, D), lambda i: (0, 0)),
            pl.BlockSpec((1, D), lambda i: (0, 0)),
        ],
        out_specs=pl.BlockSpec((_BLK, D), lambda i: (i, 0)),
        out_shape=jax.ShapeDtypeStruct((NP, D), jnp.float32),
    )(a0, a1, hp, dv, root, s, bb, wT, wb)


def _pool_body(a0_ref, a1_ref, hp_ref, dv_ref, root_ref, s_ref, bb_ref,
               bt_ref, o_ref):
    dd = dv_ref[...]
    hm = (a0_ref[...] + a1_ref[...]
          + jnp.maximum(hp_ref[...] + root_ref[...], 0.0) * (dd * dd))
    hm = hm * s_ref[...] + bb_ref[...]
    gids = lax.broadcasted_iota(jnp.int32, (G, _BLK), 0)
    onehot = (gids == bt_ref[...].reshape(1, _BLK)).astype(jnp.float32)
    contrib = jnp.dot(onehot, hm, preferred_element_type=jnp.float32)

    @pl.when(pl.program_id(0) == 0)
    def _():
        o_ref[...] = jnp.zeros_like(o_ref)

    o_ref[...] += contrib


def _tc_pool(a0, a1, hp, dv, root, s, bb, bt):
    return pl.pallas_call(
        _pool_body,
        grid=(NP // _BLK,),
        in_specs=[
            pl.BlockSpec((_BLK, D), lambda i: (i, 0)),
            pl.BlockSpec((_BLK, D), lambda i: (i, 0)),
            pl.BlockSpec((_BLK, D), lambda i: (i, 0)),
            pl.BlockSpec((_BLK, 1), lambda i: (i, 0)),
            pl.BlockSpec((1, D), lambda i: (0, 0)),
            pl.BlockSpec((1, D), lambda i: (0, 0)),
            pl.BlockSpec((1, D), lambda i: (0, 0)),
            pl.BlockSpec((_BLK, 1), lambda i: (i, 0)),
        ],
        out_specs=pl.BlockSpec((G, D), lambda i: (0, 0)),
        out_shape=jax.ShapeDtypeStruct((G, D), jnp.float32),
    )(a0, a1, hp, dv, root, s, bb, bt)


def kernel(x, edge_index, edge_attr, batch, atom_emb, lin_W, lin_b, root_emb,
           bond_e0, bond_e1, bond_e2, bn_w, bn_b):
    row2 = jnp.pad(edge_index[0], (0, EP - E), constant_values=NP - 1)
    col2 = jnp.pad(edge_index[1], (0, EP - E), constant_values=NP - 1)
    eaT = edge_attr.T
    ea0 = jnp.pad(eaT[0], (0, EP - E))
    ea1 = jnp.pad(eaT[1], (0, EP - E))
    ea2 = jnp.pad(eaT[2], (0, EP - E))
    xT = jnp.pad(x.T, ((0, 0), (0, NP - N))).reshape(9 * NP)
    aef = jnp.pad(atom_emb.reshape(9 * 119, D), ((0, 1072 - 9 * 119), (0, 0)))
    ct = (bond_e0[:, :, None, None, :] + bond_e1[:, None, :, None, :]
          + bond_e2[:, None, None, :, :]).reshape(L, 60, D)
    ct = jnp.pad(ct, ((0, 0), (0, 4), (0, 0)))
    s_bn = (bn_w / jnp.sqrt(1.0 + 1e-5)).reshape(L, 1, D)
    b_bn = bn_b.reshape(L, 1, D)
    root = root_emb.reshape(L, 1, D)
    wT = jnp.transpose(lin_W, (0, 2, 1))
    wb = lin_b.reshape(L, 1, D)
    bt = jnp.pad(batch, (0, NP - N), constant_values=G).reshape(NP, 1)

    hist2 = _hist_kernel(row2).reshape(NC, NP, 1)
    dv = _tc_dis(hist2[0], hist2[1])
    dis = dv.reshape(NP)
    norm2, cid2 = _norm_kernel(row2, col2, ea0, ea1, ea2, dis)
    h0 = _atom_kernel(xT, aef)

    hp = _tc_matmul(h0, wT[0], wb[0])
    for l in range(L):
        agg2 = _edge_kernel(hp, row2, col2, cid2, norm2, ct[l])
        if l < L - 1:
            hp = _tc_ewmm(agg2[0], agg2[1], hp, dv, root[l], s_bn[l],
                          b_bn[l], wT[l + 1], wb[l + 1])
        else:
            z = _tc_pool(agg2[0], agg2[1], hp, dv, root[l], s_bn[l],
                         b_bn[l], bt)
    return z


# speedup vs baseline: 1.6027x; 1.0436x over previous
import functools

import jax
import jax.numpy as jnp
from jax import lax
from jax.experimental import pallas as pl
from jax.experimental.pallas import tpu as pltpu
from jax.experimental.pallas import tpu_sc as plsc

N = 10000
E = 320000
D = 128
L = 3
G = 64

NC = 2
NS = 16
NW = NC * NS

NP = 10240
EP = ((E + NW * 128 - 1) // (NW * 128)) * (NW * 128)
EP = 327680
ER = EP // 128
CPW = ER // NW
HIST_PT = ER // NS
NPT = NP // NS
NPW = NP // NW

_MESH = plsc.VectorSubcoreMesh(core_axis_name="c", subcore_axis_name="s",
                               num_cores=NC, num_subcores=NS)


@functools.partial(
    pl.kernel,
    out_type=jax.ShapeDtypeStruct((NC * NP,), jnp.float32),
    mesh=_MESH,
    scratch_types=[
        pltpu.VMEM_SHARED((NP, 16), jnp.float32),
        pltpu.VMEM((128, 16), jnp.float32),
        pltpu.VMEM((128,), jnp.int32),
        pltpu.VMEM((NPT, 16), jnp.float32),
        pltpu.VMEM((NPT,), jnp.float32),
    ],
    compiler_params=pltpu.CompilerParams(needs_layout_passes=False),
)
def _hist_kernel(row2, hist_out, hist_sh, ones_v, ridx, dbuf, disv):
    core = lax.axis_index("c")
    sid = lax.axis_index("s")

    one16 = jnp.full((16,), 1.0, jnp.float32)
    zero16 = jnp.zeros((16,), jnp.float32)
    for j in range(128):
        ones_v[j, :] = one16
    for j in range(NPT):
        dbuf[j, :] = zero16
    pltpu.sync_copy(dbuf, hist_sh.at[pl.ds(sid * NPT, NPT)])
    plsc.subcore_barrier()

    def _hist(c, _):
        rr = ((core * NS + sid) * CPW + c) * 128
        pltpu.sync_copy(row2.at[pl.ds(rr, 128)], ridx)
        pltpu.sync_copy(ones_v, hist_sh.at[ridx], add=True)
        return 0
    lax.fori_loop(0, CPW, _hist, 0)
    plsc.subcore_barrier()

    pltpu.sync_copy(hist_sh.at[pl.ds(sid * NPT, NPT)], dbuf)
    zidx = jnp.zeros((16,), jnp.int32)
    for g in range(NPT // 16):
        rowi = jnp.int32(g * 16) + lax.iota(jnp.int32, 16)
        disv[pl.ds(g * 16, 16)] = plsc.load_gather(dbuf, [rowi, zidx])

    pltpu.sync_copy(disv, hist_out.at[pl.ds(core * NP + sid * NPT, NPT)])


@functools.partial(
    pl.kernel,
    out_type=(
        jax.ShapeDtypeStruct((EP,), jnp.float32),
        jax.ShapeDtypeStruct((ER * 8, 128), jnp.int32),
    ),
    mesh=_MESH,
    scratch_types=[
        pltpu.VMEM((8, 128), jnp.int32),
        pltpu.VMEM((128,), jnp.int32),
        pltpu.VMEM((128,), jnp.int32),
        pltpu.VMEM((128,), jnp.int32),
        pltpu.VMEM((128,), jnp.float32),
        pltpu.VMEM((NP,), jnp.float32),
    ],
    compiler_params=pltpu.CompilerParams(needs_layout_passes=False),
)
def _norm_kernel(row2, col2, ea0, ea1, ea2, dis, norm_out, idx3_out,
                 pk, av0, av1, av2, nmv, dis_tab):
    core = lax.axis_index("c")
    sid = lax.axis_index("s")
    w = core * NS + sid

    pltpu.sync_copy(dis, dis_tab)

    def _edge(c, _):
        rr = w * CPW + c
        sl128 = pl.ds(rr * 128, 128)
        pltpu.sync_copy(row2.at[sl128], pk.at[0])
        pltpu.sync_copy(col2.at[sl128], pk.at[1])
        pltpu.sync_copy(ea0.at[sl128], av0)
        pltpu.sync_copy(ea1.at[sl128], av1)
        pltpu.sync_copy(ea2.at[sl128], av2)
        for g in range(8):
            sl = pl.ds(g * 16, 16)
            dr = plsc.load_gather(dis_tab, [pk[0, sl]])
            dc = plsc.load_gather(dis_tab, [pk[1, sl]])
            nmv[sl] = dr * dc
            pk[2, sl] = av0[sl] * 12 + av1[sl] * 2 + av2[sl]
        pltpu.sync_copy(nmv, norm_out.at[sl128])
        pltpu.sync_copy(pk, idx3_out.at[pl.ds(rr * 8, 8)])
        return 0
    lax.fori_loop(0, CPW, _edge, 0)


@functools.partial(
    pl.kernel,
    out_type=jax.ShapeDtypeStruct((NP, D), jnp.float32),
    mesh=_MESH,
    scratch_types=[
        pltpu.VMEM((64,), jnp.int32),
        pltpu.VMEM((64,), jnp.int32),
        pltpu.VMEM((64, D), jnp.float32),
        pltpu.VMEM((64, D), jnp.float32),
        pltpu.SemaphoreType.DMA,
    ],
    compiler_params=pltpu.CompilerParams(needs_layout_passes=False),
)
def _atom_kernel(xT, aef, h0, xv, idxv, acc, gbuf, sem):
    core = lax.axis_index("c")
    sid = lax.axis_index("s")
    w = core * NS + sid

    def _chunk(c, _):
        nb = w * NPW + c * 64
        for i in range(9):
            pltpu.sync_copy(xT.at[pl.ds(i * NP + nb, 64)], xv)
            for g in range(4):
                sl = pl.ds(g * 16, 16)
                idxv[sl] = xv[sl] + jnp.int32(119 * i)
            if i == 0:
                pltpu.async_copy(aef.at[idxv], acc, sem).wait()
            else:
                pltpu.async_copy(aef.at[idxv], gbuf, sem).wait()

                def _accum(j, _):
                    for g2 in range(8):
                        sl2 = pl.ds(g2 * 16, 16)
                        acc[j, sl2] = acc[j, sl2] + gbuf[j, sl2]
                    return 0
                lax.fori_loop(0, 64, _accum, 0)
        pltpu.sync_copy(acc, h0.at[pl.ds(nb, 64)])
        return 0
    lax.fori_loop(0, NPW // 64, _chunk, 0)


@functools.partial(
    pl.kernel,
    out_type=jax.ShapeDtypeStruct((NC, NP, D), jnp.float32),
    mesh=_MESH,
    scratch_types=[
        pltpu.VMEM_SHARED((NP, D), jnp.float32),
        pltpu.VMEM((64, D), jnp.float32),
        pltpu.VMEM((8, 128), jnp.int32),
        pltpu.VMEM((8, 128), jnp.int32),
        pltpu.VMEM((128,), jnp.float32),
        pltpu.VMEM((128,), jnp.float32),
        pltpu.VMEM((128, D), jnp.float32),
        pltpu.VMEM((128, D), jnp.float32),
        pltpu.SemaphoreType.DMA,
        pltpu.SemaphoreType.DMA,
    ],
    compiler_params=pltpu.CompilerParams(needs_layout_passes=False),
)
def _edge_kernel(h, idx3, norm2, ct, agg2,
                 agg_sh, ct_v, pk0, pk1, nm0, nm1, hrow0, hrow1, sem0, sem1):
    core = lax.axis_index("c")
    sid = lax.axis_index("s")
    w = core * NS + sid

    zero16 = jnp.zeros((16,), jnp.float32)
    for j in range(128):
        for g in range(8):
            hrow0[j, pl.ds(g * 16, 16)] = zero16
    for t in range(NPT // 128):
        pltpu.sync_copy(hrow0, agg_sh.at[pl.ds(sid * NPT + t * 128, 128)])
    pltpu.sync_copy(ct, ct_v)
    plsc.subcore_barrier()

    def _load_idx(c, pk, nm):
        rr = w * CPW + c
        pltpu.sync_copy(idx3.at[pl.ds(rr * 8, 8)], pk)
        pltpu.sync_copy(norm2.at[pl.ds(rr * 128, 128)], nm)

    def _compute(pk, nm, hrow):
        for gg in range(8):
            sl = pl.ds(gg * 16, 16)
            ne16 = nm[sl]
            ce16 = pk[2, sl]
            eidx = jnp.int32(gg * 16) + lax.iota(jnp.int32, 16)

            def _col(cc, _, ne16=ne16, ce16=ce16, eidx=eidx, hrow=hrow):
                cvec = jnp.full((16,), cc, dtype=jnp.int32)
                hv = plsc.load_gather(hrow, [eidx, cvec])
                ev = plsc.load_gather(ct_v, [ce16, cvec])
                m = jnp.maximum(hv + ev, 0.0) * ne16
                plsc.store_scatter(hrow, [eidx, cvec], m)
                return 0
            lax.fori_loop(0, 128, _col, 0, unroll=8)
        pltpu.sync_copy(hrow, agg_sh.at[pk.at[1]], add=True)

    _load_idx(0, pk0, nm0)
    pltpu.async_copy(h.at[pk0.at[0]], hrow0, sem0)
    _load_idx(1, pk1, nm1)
    pltpu.async_copy(h.at[pk1.at[0]], hrow1, sem1)

    bufs = ((pk0, nm0, hrow0, sem0), (pk1, nm1, hrow1, sem1))

    def _pair(it, _):
        for b in range(2):
            pk, nm, hrow, sem = bufs[b]
            c = it * 2 + b
            pltpu.make_async_copy(h.at[pk.at[0]], hrow, sem).wait()
            _compute(pk, nm, hrow)

            @pl.when(c + 2 < CPW)
            def _(pk=pk, nm=nm, hrow=hrow, sem=sem, c=c):
                _load_idx(c + 2, pk, nm)
                pltpu.async_copy(h.at[pk.at[0]], hrow, sem)
            return_val = 0
        return 0
    lax.fori_loop(0, CPW // 2, _pair, 0)
    plsc.subcore_barrier()

    pltpu.sync_copy(agg_sh.at[pl.ds(sid * NPT, NPT)],
                    agg2.at[core].at[pl.ds(sid * NPT, NPT)])


_BLK = 1024


def _dis_body(h0_ref, h1_ref, o_ref):
    o_ref[...] = lax.rsqrt(h0_ref[...] + h1_ref[...] + 1.0)


def _tc_dis(hist0, hist1):
    return pl.pallas_call(
        _dis_body,
        out_shape=jax.ShapeDtypeStruct((NP, 1), jnp.float32),
    )(hist0, hist1)


def _mm_body(x_ref, w_ref, b_ref, o_ref):
    o_ref[...] = jnp.dot(x_ref[...], w_ref[...],
                         preferred_element_type=jnp.float32) + b_ref[...]


def _tc_matmul(hx, wT, b):
    return pl.pallas_call(
        _mm_body,
        grid=(NP // _BLK,),
        in_specs=[
            pl.BlockSpec((_BLK, D), lambda i: (i, 0)),
            pl.BlockSpec((D, D), lambda i: (0, 0)),
            pl.BlockSpec((1, D), lambda i: (0, 0)),
        ],
        out_specs=pl.BlockSpec((_BLK, D), lambda i: (i, 0)),
        out_shape=jax.ShapeDtypeStruct((NP, D), jnp.float32),
    )(hx, wT, b)


def _ewmm_body(a0_ref, a1_ref, hp_ref, dv_ref, root_ref, s_ref, bb_ref,
               w_ref, wb_ref, o_ref):
    dd = dv_ref[...]
    hm = (a0_ref[...] + a1_ref[...]
          + jnp.maximum(hp_ref[...] + root_ref[...], 0.0) * (dd * dd))
    hm = hm * s_ref[...] + bb_ref[...]
    hm = jnp.maximum(hm, 0.0)
    o_ref[...] = jnp.dot(hm, w_ref[...],
                         preferred_element_type=jnp.float32) + wb_ref[...]


def _tc_ewmm(a0, a1, hp, dv, root, s, bb, wT, wb):
    return pl.pallas_call(
        _ewmm_body,
        grid=(NP // _BLK,),
        in_specs=[
            pl.BlockSpec((_BLK, D), lambda i: (i, 0)),
            pl.BlockSpec((_BLK, D), lambda i: (i, 0)),
            pl.BlockSpec((_BLK, D), lambda i: (i, 0)),
            pl.BlockSpec((_BLK, 1), lambda i: (i, 0)),
            pl.BlockSpec((1, D), lambda i: (0, 0)),
            pl.BlockSpec((1, D), lambda i: (0, 0)),
            pl.BlockSpec((1, D), lambda i: (0, 0)),
            pl.BlockSpec((D, D), lambda i: (0, 0)),
            pl.BlockSpec((1, D), lambda i: (0, 0)),
        ],
        out_specs=pl.BlockSpec((_BLK, D), lambda i: (i, 0)),
        out_shape=jax.ShapeDtypeStruct((NP, D), jnp.float32),
    )(a0, a1, hp, dv, root, s, bb, wT, wb)


def _pool_body(a0_ref, a1_ref, hp_ref, dv_ref, root_ref, s_ref, bb_ref,
               bt_ref, o_ref):
    dd = dv_ref[...]
    hm = (a0_ref[...] + a1_ref[...]
          + jnp.maximum(hp_ref[...] + root_ref[...], 0.0) * (dd * dd))
    hm = hm * s_ref[...] + bb_ref[...]
    gids = lax.broadcasted_iota(jnp.int32, (G, _BLK), 0)
    onehot = (gids == bt_ref[...].reshape(1, _BLK)).astype(jnp.float32)
    contrib = jnp.dot(onehot, hm, preferred_element_type=jnp.float32)

    @pl.when(pl.program_id(0) == 0)
    def _():
        o_ref[...] = jnp.zeros_like(o_ref)

    o_ref[...] += contrib


def _tc_pool(a0, a1, hp, dv, root, s, bb, bt):
    return pl.pallas_call(
        _pool_body,
        grid=(NP // _BLK,),
        in_specs=[
            pl.BlockSpec((_BLK, D), lambda i: (i, 0)),
            pl.BlockSpec((_BLK, D), lambda i: (i, 0)),
            pl.BlockSpec((_BLK, D), lambda i: (i, 0)),
            pl.BlockSpec((_BLK, 1), lambda i: (i, 0)),
            pl.BlockSpec((1, D), lambda i: (0, 0)),
            pl.BlockSpec((1, D), lambda i: (0, 0)),
            pl.BlockSpec((1, D), lambda i: (0, 0)),
            pl.BlockSpec((_BLK, 1), lambda i: (i, 0)),
        ],
        out_specs=pl.BlockSpec((G, D), lambda i: (0, 0)),
        out_shape=jax.ShapeDtypeStruct((G, D), jnp.float32),
    )(a0, a1, hp, dv, root, s, bb, bt)


def kernel(x, edge_index, edge_attr, batch, atom_emb, lin_W, lin_b, root_emb,
           bond_e0, bond_e1, bond_e2, bn_w, bn_b):
    row2 = jnp.pad(edge_index[0], (0, EP - E), constant_values=NP - 1)
    col2 = jnp.pad(edge_index[1], (0, EP - E), constant_values=NP - 1)
    eaT = edge_attr.T
    ea0 = jnp.pad(eaT[0], (0, EP - E))
    ea1 = jnp.pad(eaT[1], (0, EP - E))
    ea2 = jnp.pad(eaT[2], (0, EP - E))
    xT = jnp.pad(x.T, ((0, 0), (0, NP - N))).reshape(9 * NP)
    aef = jnp.pad(atom_emb.reshape(9 * 119, D), ((0, 1072 - 9 * 119), (0, 0)))
    ct = (bond_e0[:, :, None, None, :] + bond_e1[:, None, :, None, :]
          + bond_e2[:, None, None, :, :]).reshape(L, 60, D)
    ct = jnp.pad(ct, ((0, 0), (0, 4), (0, 0)))
    s_bn = (bn_w / jnp.sqrt(1.0 + 1e-5)).reshape(L, 1, D)
    b_bn = bn_b.reshape(L, 1, D)
    root = root_emb.reshape(L, 1, D)
    wT = jnp.transpose(lin_W, (0, 2, 1))
    wb = lin_b.reshape(L, 1, D)
    bt = jnp.pad(batch, (0, NP - N), constant_values=G).reshape(NP, 1)

    hist2 = _hist_kernel(row2).reshape(NC, NP, 1)
    dv = _tc_dis(hist2[0], hist2[1])
    dis = dv.reshape(NP)
    norm2, idx3 = _norm_kernel(row2, col2, ea0, ea1, ea2, dis)
    h0 = _atom_kernel(xT, aef)

    hp = _tc_matmul(h0, wT[0], wb[0])
    for l in range(L):
        agg2 = _edge_kernel(hp, idx3, norm2, ct[l])
        if l < L - 1:
            hp = _tc_ewmm(agg2[0], agg2[1], hp, dv, root[l], s_bn[l],
                          b_bn[l], wT[l + 1], wb[l + 1])
        else:
            z = _tc_pool(agg2[0], agg2[1], hp, dv, root[l], s_bn[l],
                         b_bn[l], bt)
    return z


# speedup vs baseline: 2.4245x; 1.5128x over previous
import functools

import jax
import jax.numpy as jnp
from jax import lax
from jax.experimental import pallas as pl
from jax.experimental.pallas import tpu as pltpu
from jax.experimental.pallas import tpu_sc as plsc

N = 10000
E = 320000
D = 128
L = 3
G = 64

NC = 2
NS = 16
NW = NC * NS

NP = 10240
EP = ((E + NW * 128 - 1) // (NW * 128)) * (NW * 128)
EP = 327680
ER = EP // 128
CPW = ER // NW
HIST_PT = ER // NS
NPT = NP // NS
NPW = NP // NW

_MESH = plsc.VectorSubcoreMesh(core_axis_name="c", subcore_axis_name="s",
                               num_cores=NC, num_subcores=NS)


@functools.partial(
    pl.kernel,
    out_type=jax.ShapeDtypeStruct((NC * NP,), jnp.float32),
    mesh=_MESH,
    scratch_types=[
        pltpu.VMEM_SHARED((NP, 16), jnp.float32),
        pltpu.VMEM((128, 16), jnp.float32),
        pltpu.VMEM((128,), jnp.int32),
        pltpu.VMEM((NPT, 16), jnp.float32),
        pltpu.VMEM((NPT,), jnp.float32),
    ],
    compiler_params=pltpu.CompilerParams(needs_layout_passes=False),
)
def _hist_kernel(row2, hist_out, hist_sh, ones_v, ridx, dbuf, disv):
    core = lax.axis_index("c")
    sid = lax.axis_index("s")

    one16 = jnp.full((16,), 1.0, jnp.float32)
    zero16 = jnp.zeros((16,), jnp.float32)
    for j in range(128):
        ones_v[j, :] = one16
    for j in range(NPT):
        dbuf[j, :] = zero16
    pltpu.sync_copy(dbuf, hist_sh.at[pl.ds(sid * NPT, NPT)])
    plsc.subcore_barrier()

    def _hist(c, _):
        rr = ((core * NS + sid) * CPW + c) * 128
        pltpu.sync_copy(row2.at[pl.ds(rr, 128)], ridx)
        pltpu.sync_copy(ones_v, hist_sh.at[ridx], add=True)
        return 0
    lax.fori_loop(0, CPW, _hist, 0)
    plsc.subcore_barrier()

    pltpu.sync_copy(hist_sh.at[pl.ds(sid * NPT, NPT)], dbuf)
    zidx = jnp.zeros((16,), jnp.int32)
    for g in range(NPT // 16):
        rowi = jnp.int32(g * 16) + lax.iota(jnp.int32, 16)
        disv[pl.ds(g * 16, 16)] = plsc.load_gather(dbuf, [rowi, zidx])

    pltpu.sync_copy(disv, hist_out.at[pl.ds(core * NP + sid * NPT, NPT)])


@functools.partial(
    pl.kernel,
    out_type=(
        jax.ShapeDtypeStruct((EP,), jnp.float32),
        jax.ShapeDtypeStruct((ER * 8, 128), jnp.int32),
    ),
    mesh=_MESH,
    scratch_types=[
        pltpu.VMEM((8, 128), jnp.int32),
        pltpu.VMEM((128,), jnp.int32),
        pltpu.VMEM((128,), jnp.int32),
        pltpu.VMEM((128,), jnp.int32),
        pltpu.VMEM((128,), jnp.float32),
        pltpu.VMEM((NP,), jnp.float32),
    ],
    compiler_params=pltpu.CompilerParams(needs_layout_passes=False),
)
def _norm_kernel(row2, col2, ea0, ea1, ea2, dis, norm_out, idx3_out,
                 pk, av0, av1, av2, nmv, dis_tab):
    core = lax.axis_index("c")
    sid = lax.axis_index("s")
    w = core * NS + sid

    pltpu.sync_copy(dis, dis_tab)

    def _edge(c, _):
        rr = w * CPW + c
        sl128 = pl.ds(rr * 128, 128)
        pltpu.sync_copy(row2.at[sl128], pk.at[0])
        pltpu.sync_copy(col2.at[sl128], pk.at[1])
        pltpu.sync_copy(ea0.at[sl128], av0)
        pltpu.sync_copy(ea1.at[sl128], av1)
        pltpu.sync_copy(ea2.at[sl128], av2)
        for g in range(8):
            sl = pl.ds(g * 16, 16)
            dr = plsc.load_gather(dis_tab, [pk[0, sl]])
            dc = plsc.load_gather(dis_tab, [pk[1, sl]])
            nmv[sl] = dr * dc
            pk[2, sl] = av0[sl] * 12 + av1[sl] * 2 + av2[sl]
        pltpu.sync_copy(nmv, norm_out.at[sl128])
        pltpu.sync_copy(pk, idx3_out.at[pl.ds(rr * 8, 8)])
        return 0
    lax.fori_loop(0, CPW, _edge, 0)


@functools.partial(
    pl.kernel,
    out_type=jax.ShapeDtypeStruct((NP, D), jnp.float32),
    mesh=_MESH,
    scratch_types=[
        pltpu.VMEM((64,), jnp.int32),
        pltpu.VMEM((64,), jnp.int32),
        pltpu.VMEM((64, D), jnp.float32),
        pltpu.VMEM((64, D), jnp.float32),
        pltpu.SemaphoreType.DMA,
    ],
    compiler_params=pltpu.CompilerParams(needs_layout_passes=False),
)
def _atom_kernel(xT, aef, h0, xv, idxv, acc, gbuf, sem):
    core = lax.axis_index("c")
    sid = lax.axis_index("s")
    w = core * NS + sid

    def _chunk(c, _):
        nb = w * NPW + c * 64
        for i in range(9):
            pltpu.sync_copy(xT.at[pl.ds(i * NP + nb, 64)], xv)
            for g in range(4):
                sl = pl.ds(g * 16, 16)
                idxv[sl] = xv[sl] + jnp.int32(119 * i)
            if i == 0:
                pltpu.async_copy(aef.at[idxv], acc, sem).wait()
            else:
                pltpu.async_copy(aef.at[idxv], gbuf, sem).wait()

                def _accum(j, _):
                    for g2 in range(8):
                        sl2 = pl.ds(g2 * 16, 16)
                        acc[j, sl2] = acc[j, sl2] + gbuf[j, sl2]
                    return 0
                lax.fori_loop(0, 64, _accum, 0)
        pltpu.sync_copy(acc, h0.at[pl.ds(nb, 64)])
        return 0
    lax.fori_loop(0, NPW // 64, _chunk, 0)


@functools.partial(
    pl.kernel,
    out_type=jax.ShapeDtypeStruct((NC, NP, D), jnp.float32),
    mesh=_MESH,
    scratch_types=[
        pltpu.VMEM_SHARED((NP, D), jnp.float32),
        pltpu.VMEM((64, D), jnp.float32),
        pltpu.VMEM((8, 128), jnp.int32),
        pltpu.VMEM((128,), jnp.float32),
        pltpu.VMEM((128, D), jnp.float32),
        pltpu.VMEM((128, D), jnp.float32),
        pltpu.SemaphoreType.DMA,
    ],
    compiler_params=pltpu.CompilerParams(needs_layout_passes=False),
)
def _edge_kernel(h, idx3, norm2, ct, agg2,
                 agg_sh, ct_v, pk0, nm0, hrow0, msg0, sem0):
    core = lax.axis_index("c")
    sid = lax.axis_index("s")
    w = core * NS + sid

    zero16 = jnp.zeros((16,), jnp.float32)
    for j in range(128):
        for g in range(8):
            hrow0[j, pl.ds(g * 16, 16)] = zero16
    for t in range(NPT // 128):
        pltpu.sync_copy(hrow0, agg_sh.at[pl.ds(sid * NPT + t * 128, 128)])
    pltpu.sync_copy(ct, ct_v)
    plsc.subcore_barrier()

    def _load_idx(c, pk, nm):
        rr = w * CPW + c
        pltpu.sync_copy(idx3.at[pl.ds(rr * 8, 8)], pk)
        pltpu.sync_copy(norm2.at[pl.ds(rr * 128, 128)], nm)

    def _compute(pk, nm, hrow, msg):
        for gg in range(8):
            sl = pl.ds(gg * 16, 16)
            ne16 = nm[sl]
            ce16 = pk[2, sl]
            eidx = jnp.int32(gg * 16) + lax.iota(jnp.int32, 16)

            @plsc.parallel_loop(0, 128, unroll=8)
            def _col(cc, ne16=ne16, ce16=ce16, eidx=eidx,
                     hrow=hrow, msg=msg):
                cvec = jnp.full((16,), cc, dtype=jnp.int32)
                hv = plsc.load_gather(hrow, [eidx, cvec])
                ev = plsc.load_gather(ct_v, [ce16, cvec])
                m = jnp.maximum(hv + ev, 0.0) * ne16
                plsc.store_scatter(hrow, [eidx, cvec], m)
        pltpu.sync_copy(hrow, agg_sh.at[pk.at[1]], add=True)

    def _chunk(c, _):
        _load_idx(c, pk0, nm0)
        pltpu.async_copy(h.at[pk0.at[0]], hrow0, sem0).wait()
        _compute(pk0, nm0, hrow0, msg0)
        return 0
    lax.fori_loop(0, CPW, _chunk, 0)
    plsc.subcore_barrier()

    pltpu.sync_copy(agg_sh.at[pl.ds(sid * NPT, NPT)],
                    agg2.at[core].at[pl.ds(sid * NPT, NPT)])


_BLK = 1024


def _dis_body(h0_ref, h1_ref, o_ref):
    o_ref[...] = lax.rsqrt(h0_ref[...] + h1_ref[...] + 1.0)


def _tc_dis(hist0, hist1):
    return pl.pallas_call(
        _dis_body,
        out_shape=jax.ShapeDtypeStruct((NP, 1), jnp.float32),
    )(hist0, hist1)


def _mm_body(x_ref, w_ref, b_ref, o_ref):
    o_ref[...] = jnp.dot(x_ref[...], w_ref[...],
                         preferred_element_type=jnp.float32) + b_ref[...]


def _tc_matmul(hx, wT, b):
    return pl.pallas_call(
        _mm_body,
        grid=(NP // _BLK,),
        in_specs=[
            pl.BlockSpec((_BLK, D), lambda i: (i, 0)),
            pl.BlockSpec((D, D), lambda i: (0, 0)),
            pl.BlockSpec((1, D), lambda i: (0, 0)),
        ],
        out_specs=pl.BlockSpec((_BLK, D), lambda i: (i, 0)),
        out_shape=jax.ShapeDtypeStruct((NP, D), jnp.float32),
    )(hx, wT, b)


def _ewmm_body(a0_ref, a1_ref, hp_ref, dv_ref, root_ref, s_ref, bb_ref,
               w_ref, wb_ref, o_ref):
    dd = dv_ref[...]
    hm = (a0_ref[...] + a1_ref[...]
          + jnp.maximum(hp_ref[...] + root_ref[...], 0.0) * (dd * dd))
    hm = hm * s_ref[...] + bb_ref[...]
    hm = jnp.maximum(hm, 0.0)
    o_ref[...] = jnp.dot(hm, w_ref[...],
                         preferred_element_type=jnp.float32) + wb_ref[...]


def _tc_ewmm(a0, a1, hp, dv, root, s, bb, wT, wb):
    return pl.pallas_call(
        _ewmm_body,
        grid=(NP // _BLK,),
        in_specs=[
            pl.BlockSpec((_BLK, D), lambda i: (i, 0)),
            pl.BlockSpec((_BLK, D), lambda i: (i, 0)),
            pl.BlockSpec((_BLK, D), lambda i: (i, 0)),
            pl.BlockSpec((_BLK, 1), lambda i: (i, 0)),
            pl.BlockSpec((1, D), lambda i: (0, 0)),
            pl.BlockSpec((1, D), lambda i: (0, 0)),
            pl.BlockSpec((1, D), lambda i: (0, 0)),
            pl.BlockSpec((D, D), lambda i: (0, 0)),
            pl.BlockSpec((1, D), lambda i: (0, 0)),
        ],
        out_specs=pl.BlockSpec((_BLK, D), lambda i: (i, 0)),
        out_shape=jax.ShapeDtypeStruct((NP, D), jnp.float32),
    )(a0, a1, hp, dv, root, s, bb, wT, wb)


def _pool_body(a0_ref, a1_ref, hp_ref, dv_ref, root_ref, s_ref, bb_ref,
               bt_ref, o_ref):
    dd = dv_ref[...]
    hm = (a0_ref[...] + a1_ref[...]
          + jnp.maximum(hp_ref[...] + root_ref[...], 0.0) * (dd * dd))
    hm = hm * s_ref[...] + bb_ref[...]
    gids = lax.broadcasted_iota(jnp.int32, (G, _BLK), 0)
    onehot = (gids == bt_ref[...].reshape(1, _BLK)).astype(jnp.float32)
    contrib = jnp.dot(onehot, hm, preferred_element_type=jnp.float32)

    @pl.when(pl.program_id(0) == 0)
    def _():
        o_ref[...] = jnp.zeros_like(o_ref)

    o_ref[...] += contrib


def _tc_pool(a0, a1, hp, dv, root, s, bb, bt):
    return pl.pallas_call(
        _pool_body,
        grid=(NP // _BLK,),
        in_specs=[
            pl.BlockSpec((_BLK, D), lambda i: (i, 0)),
            pl.BlockSpec((_BLK, D), lambda i: (i, 0)),
            pl.BlockSpec((_BLK, D), lambda i: (i, 0)),
            pl.BlockSpec((_BLK, 1), lambda i: (i, 0)),
            pl.BlockSpec((1, D), lambda i: (0, 0)),
            pl.BlockSpec((1, D), lambda i: (0, 0)),
            pl.BlockSpec((1, D), lambda i: (0, 0)),
            pl.BlockSpec((_BLK, 1), lambda i: (i, 0)),
        ],
        out_specs=pl.BlockSpec((G, D), lambda i: (0, 0)),
        out_shape=jax.ShapeDtypeStruct((G, D), jnp.float32),
    )(a0, a1, hp, dv, root, s, bb, bt)


def kernel(x, edge_index, edge_attr, batch, atom_emb, lin_W, lin_b, root_emb,
           bond_e0, bond_e1, bond_e2, bn_w, bn_b):
    row2 = jnp.pad(edge_index[0], (0, EP - E), constant_values=NP - 1)
    col2 = jnp.pad(edge_index[1], (0, EP - E), constant_values=NP - 1)
    eaT = edge_attr.T
    ea0 = jnp.pad(eaT[0], (0, EP - E))
    ea1 = jnp.pad(eaT[1], (0, EP - E))
    ea2 = jnp.pad(eaT[2], (0, EP - E))
    xT = jnp.pad(x.T, ((0, 0), (0, NP - N))).reshape(9 * NP)
    aef = jnp.pad(atom_emb.reshape(9 * 119, D), ((0, 1072 - 9 * 119), (0, 0)))
    ct = (bond_e0[:, :, None, None, :] + bond_e1[:, None, :, None, :]
          + bond_e2[:, None, None, :, :]).reshape(L, 60, D)
    ct = jnp.pad(ct, ((0, 0), (0, 4), (0, 0)))
    s_bn = (bn_w / jnp.sqrt(1.0 + 1e-5)).reshape(L, 1, D)
    b_bn = bn_b.reshape(L, 1, D)
    root = root_emb.reshape(L, 1, D)
    wT = jnp.transpose(lin_W, (0, 2, 1))
    wb = lin_b.reshape(L, 1, D)
    bt = jnp.pad(batch, (0, NP - N), constant_values=G).reshape(NP, 1)

    hist2 = _hist_kernel(row2).reshape(NC, NP, 1)
    dv = _tc_dis(hist2[0], hist2[1])
    dis = dv.reshape(NP)
    norm2, idx3 = _norm_kernel(row2, col2, ea0, ea1, ea2, dis)
    h0 = _atom_kernel(xT, aef)

    hp = _tc_matmul(h0, wT[0], wb[0])
    for l in range(L):
        agg2 = _edge_kernel(hp, idx3, norm2, ct[l])
        if l < L - 1:
            hp = _tc_ewmm(agg2[0], agg2[1], hp, dv, root[l], s_bn[l],
                          b_bn[l], wT[l + 1], wb[l + 1])
        else:
            z = _tc_pool(agg2[0], agg2[1], hp, dv, root[l], s_bn[l],
                         b_bn[l], bt)
    return z


# speedup vs baseline: 2.5261x; 1.0419x over previous
import functools

import jax
import jax.numpy as jnp
from jax import lax
from jax.experimental import pallas as pl
from jax.experimental.pallas import tpu as pltpu
from jax.experimental.pallas import tpu_sc as plsc

N = 10000
E = 320000
D = 128
L = 3
G = 64

NC = 2
NS = 16
NW = NC * NS

NP = 10240
EP = ((E + NW * 128 - 1) // (NW * 128)) * (NW * 128)
EP = 327680
ER = EP // 128
CPW = ER // NW
HIST_PT = ER // NS
NPT = NP // NS
NPW = NP // NW

_MESH = plsc.VectorSubcoreMesh(core_axis_name="c", subcore_axis_name="s",
                               num_cores=NC, num_subcores=NS)


@functools.partial(
    pl.kernel,
    out_type=jax.ShapeDtypeStruct((NC * NP,), jnp.float32),
    mesh=_MESH,
    scratch_types=[
        pltpu.VMEM_SHARED((NP, 16), jnp.float32),
        pltpu.VMEM((128, 16), jnp.float32),
        pltpu.VMEM((128,), jnp.int32),
        pltpu.VMEM((NPT, 16), jnp.float32),
        pltpu.VMEM((NPT,), jnp.float32),
    ],
    compiler_params=pltpu.CompilerParams(needs_layout_passes=False),
)
def _hist_kernel(row2, hist_out, hist_sh, ones_v, ridx, dbuf, disv):
    core = lax.axis_index("c")
    sid = lax.axis_index("s")

    one16 = jnp.full((16,), 1.0, jnp.float32)
    zero16 = jnp.zeros((16,), jnp.float32)
    for j in range(128):
        ones_v[j, :] = one16
    for j in range(NPT):
        dbuf[j, :] = zero16
    pltpu.sync_copy(dbuf, hist_sh.at[pl.ds(sid * NPT, NPT)])
    plsc.subcore_barrier()

    def _hist(c, _):
        rr = ((core * NS + sid) * CPW + c) * 128
        pltpu.sync_copy(row2.at[pl.ds(rr, 128)], ridx)
        pltpu.sync_copy(ones_v, hist_sh.at[ridx], add=True)
        return 0
    lax.fori_loop(0, CPW, _hist, 0)
    plsc.subcore_barrier()

    pltpu.sync_copy(hist_sh.at[pl.ds(sid * NPT, NPT)], dbuf)
    zidx = jnp.zeros((16,), jnp.int32)
    for g in range(NPT // 16):
        rowi = jnp.int32(g * 16) + lax.iota(jnp.int32, 16)
        disv[pl.ds(g * 16, 16)] = plsc.load_gather(dbuf, [rowi, zidx])

    pltpu.sync_copy(disv, hist_out.at[pl.ds(core * NP + sid * NPT, NPT)])


@functools.partial(
    pl.kernel,
    out_type=(
        jax.ShapeDtypeStruct((EP,), jnp.float32),
        jax.ShapeDtypeStruct((ER * 8, 128), jnp.int32),
    ),
    mesh=_MESH,
    scratch_types=[
        pltpu.VMEM((8, 128), jnp.int32),
        pltpu.VMEM((128,), jnp.int32),
        pltpu.VMEM((128,), jnp.int32),
        pltpu.VMEM((128,), jnp.int32),
        pltpu.VMEM((128,), jnp.float32),
        pltpu.VMEM((NP,), jnp.float32),
    ],
    compiler_params=pltpu.CompilerParams(needs_layout_passes=False),
)
def _norm_kernel(row2, col2, ea0, ea1, ea2, dis, norm_out, idx3_out,
                 pk, av0, av1, av2, nmv, dis_tab):
    core = lax.axis_index("c")
    sid = lax.axis_index("s")
    w = core * NS + sid

    pltpu.sync_copy(dis, dis_tab)

    def _edge(c, _):
        rr = w * CPW + c
        sl128 = pl.ds(rr * 128, 128)
        pltpu.sync_copy(row2.at[sl128], pk.at[0])
        pltpu.sync_copy(col2.at[sl128], pk.at[1])
        pltpu.sync_copy(ea0.at[sl128], av0)
        pltpu.sync_copy(ea1.at[sl128], av1)
        pltpu.sync_copy(ea2.at[sl128], av2)
        for g in range(8):
            sl = pl.ds(g * 16, 16)
            dr = plsc.load_gather(dis_tab, [pk[0, sl]])
            dc = plsc.load_gather(dis_tab, [pk[1, sl]])
            nmv[sl] = dr * dc
            pk[2, sl] = av0[sl] * 12 + av1[sl] * 2 + av2[sl]
        pltpu.sync_copy(nmv, norm_out.at[sl128])
        pltpu.sync_copy(pk, idx3_out.at[pl.ds(rr * 8, 8)])
        return 0
    lax.fori_loop(0, CPW, _edge, 0)


@functools.partial(
    pl.kernel,
    out_type=jax.ShapeDtypeStruct((NP, D), jnp.float32),
    mesh=_MESH,
    scratch_types=[
        pltpu.VMEM((64,), jnp.int32),
        pltpu.VMEM((64,), jnp.int32),
        pltpu.VMEM((64, D), jnp.float32),
        pltpu.VMEM((64, D), jnp.float32),
        pltpu.SemaphoreType.DMA,
    ],
    compiler_params=pltpu.CompilerParams(needs_layout_passes=False),
)
def _atom_kernel(xT, aef, h0, xv, idxv, acc, gbuf, sem):
    core = lax.axis_index("c")
    sid = lax.axis_index("s")
    w = core * NS + sid

    def _chunk(c, _):
        nb = w * NPW + c * 64
        for i in range(9):
            pltpu.sync_copy(xT.at[pl.ds(i * NP + nb, 64)], xv)
            for g in range(4):
                sl = pl.ds(g * 16, 16)
                idxv[sl] = xv[sl] + jnp.int32(119 * i)
            if i == 0:
                pltpu.async_copy(aef.at[idxv], acc, sem).wait()
            else:
                pltpu.async_copy(aef.at[idxv], gbuf, sem).wait()

                def _accum(j, _):
                    for g2 in range(8):
                        sl2 = pl.ds(g2 * 16, 16)
                        acc[j, sl2] = acc[j, sl2] + gbuf[j, sl2]
                    return 0
                lax.fori_loop(0, 64, _accum, 0)
        pltpu.sync_copy(acc, h0.at[pl.ds(nb, 64)])
        return 0
    lax.fori_loop(0, NPW // 64, _chunk, 0)


@functools.partial(
    pl.kernel,
    out_type=jax.ShapeDtypeStruct((NC, NP, D), jnp.float32),
    mesh=_MESH,
    scratch_types=[
        pltpu.VMEM_SHARED((NP, D), jnp.float32),
        pltpu.VMEM((64, D), jnp.float32),
        pltpu.VMEM((8, 128), jnp.int32),
        pltpu.VMEM((8, 128), jnp.int32),
        pltpu.VMEM((128,), jnp.float32),
        pltpu.VMEM((128,), jnp.float32),
        pltpu.VMEM((128, D), jnp.float32),
        pltpu.VMEM((128, D), jnp.float32),
        pltpu.SemaphoreType.DMA,
        pltpu.SemaphoreType.DMA,
    ],
    compiler_params=pltpu.CompilerParams(needs_layout_passes=False),
)
def _edge_kernel(h, idx3, norm2, ct, agg2,
                 agg_sh, ct_v, pk0, pk1, nm0, nm1, hrow0, hrow1, sem0, sem1):
    core = lax.axis_index("c")
    sid = lax.axis_index("s")
    w = core * NS + sid

    zero16 = jnp.zeros((16,), jnp.float32)
    for j in range(128):
        for g in range(8):
            hrow0[j, pl.ds(g * 16, 16)] = zero16
    for t in range(NPT // 128):
        pltpu.sync_copy(hrow0, agg_sh.at[pl.ds(sid * NPT + t * 128, 128)])
    pltpu.sync_copy(ct, ct_v)
    plsc.subcore_barrier()

    def _load_idx(c, pk, nm):
        rr = w * CPW + c
        pltpu.sync_copy(idx3.at[pl.ds(rr * 8, 8)], pk)
        pltpu.sync_copy(norm2.at[pl.ds(rr * 128, 128)], nm)

    def _compute(pk, nm, hrow, msg):
        for gg in range(8):
            sl = pl.ds(gg * 16, 16)
            ne16 = nm[sl]
            ce16 = pk[2, sl]
            eidx = jnp.int32(gg * 16) + lax.iota(jnp.int32, 16)

            @plsc.parallel_loop(0, 128, unroll=8)
            def _col(cc, ne16=ne16, ce16=ce16, eidx=eidx,
                     hrow=hrow, msg=msg):
                cvec = jnp.full((16,), cc, dtype=jnp.int32)
                hv = plsc.load_gather(hrow, [eidx, cvec])
                ev = plsc.load_gather(ct_v, [ce16, cvec])
                m = jnp.maximum(hv + ev, 0.0) * ne16
                plsc.store_scatter(hrow, [eidx, cvec], m)
        pltpu.sync_copy(hrow, agg_sh.at[pk.at[1]], add=True)

    _load_idx(0, pk0, nm0)
    pltpu.async_copy(h.at[pk0.at[0]], hrow0, sem0)
    _load_idx(1, pk1, nm1)
    pltpu.async_copy(h.at[pk1.at[0]], hrow1, sem1)

    bufs = ((pk0, nm0, hrow0, sem0), (pk1, nm1, hrow1, sem1))

    def _pair(it, _):
        for b in range(2):
            pk, nm, hrow, sem = bufs[b]
            c = it * 2 + b
            pltpu.make_async_copy(h.at[pk.at[0]], hrow, sem).wait()
            _compute(pk, nm, hrow, hrow)

            @pl.when(c + 2 < CPW)
            def _(pk=pk, nm=nm, hrow=hrow, sem=sem, c=c):
                _load_idx(c + 2, pk, nm)
                pltpu.async_copy(h.at[pk.at[0]], hrow, sem)
        return 0
    lax.fori_loop(0, CPW // 2, _pair, 0)
    plsc.subcore_barrier()

    pltpu.sync_copy(agg_sh.at[pl.ds(sid * NPT, NPT)],
                    agg2.at[core].at[pl.ds(sid * NPT, NPT)])


_BLK = 1024


def _dis_body(h0_ref, h1_ref, o_ref):
    o_ref[...] = lax.rsqrt(h0_ref[...] + h1_ref[...] + 1.0)


def _tc_dis(hist0, hist1):
    return pl.pallas_call(
        _dis_body,
        out_shape=jax.ShapeDtypeStruct((NP, 1), jnp.float32),
    )(hist0, hist1)


def _mm_body(x_ref, w_ref, b_ref, o_ref):
    o_ref[...] = jnp.dot(x_ref[...], w_ref[...],
                         preferred_element_type=jnp.float32) + b_ref[...]


def _tc_matmul(hx, wT, b):
    return pl.pallas_call(
        _mm_body,
        grid=(NP // _BLK,),
        in_specs=[
            pl.BlockSpec((_BLK, D), lambda i: (i, 0)),
            pl.BlockSpec((D, D), lambda i: (0, 0)),
            pl.BlockSpec((1, D), lambda i: (0, 0)),
        ],
        out_specs=pl.BlockSpec((_BLK, D), lambda i: (i, 0)),
        out_shape=jax.ShapeDtypeStruct((NP, D), jnp.float32),
    )(hx, wT, b)


def _ewmm_body(a0_ref, a1_ref, hp_ref, dv_ref, root_ref, s_ref, bb_ref,
               w_ref, wb_ref, o_ref):
    dd = dv_ref[...]
    hm = (a0_ref[...] + a1_ref[...]
          + jnp.maximum(hp_ref[...] + root_ref[...], 0.0) * (dd * dd))
    hm = hm * s_ref[...] + bb_ref[...]
    hm = jnp.maximum(hm, 0.0)
    o_ref[...] = jnp.dot(hm, w_ref[...],
                         preferred_element_type=jnp.float32) + wb_ref[...]


def _tc_ewmm(a0, a1, hp, dv, root, s, bb, wT, wb):
    return pl.pallas_call(
        _ewmm_body,
        grid=(NP // _BLK,),
        in_specs=[
            pl.BlockSpec((_BLK, D), lambda i: (i, 0)),
            pl.BlockSpec((_BLK, D), lambda i: (i, 0)),
            pl.BlockSpec((_BLK, D), lambda i: (i, 0)),
            pl.BlockSpec((_BLK, 1), lambda i: (i, 0)),
            pl.BlockSpec((1, D), lambda i: (0, 0)),
            pl.BlockSpec((1, D), lambda i: (0, 0)),
            pl.BlockSpec((1, D), lambda i: (0, 0)),
            pl.BlockSpec((D, D), lambda i: (0, 0)),
            pl.BlockSpec((1, D), lambda i: (0, 0)),
        ],
        out_specs=pl.BlockSpec((_BLK, D), lambda i: (i, 0)),
        out_shape=jax.ShapeDtypeStruct((NP, D), jnp.float32),
    )(a0, a1, hp, dv, root, s, bb, wT, wb)


def _pool_body(a0_ref, a1_ref, hp_ref, dv_ref, root_ref, s_ref, bb_ref,
               bt_ref, o_ref):
    dd = dv_ref[...]
    hm = (a0_ref[...] + a1_ref[...]
          + jnp.maximum(hp_ref[...] + root_ref[...], 0.0) * (dd * dd))
    hm = hm * s_ref[...] + bb_ref[...]
    gids = lax.broadcasted_iota(jnp.int32, (G, _BLK), 0)
    onehot = (gids == bt_ref[...].reshape(1, _BLK)).astype(jnp.float32)
    contrib = jnp.dot(onehot, hm, preferred_element_type=jnp.float32)

    @pl.when(pl.program_id(0) == 0)
    def _():
        o_ref[...] = jnp.zeros_like(o_ref)

    o_ref[...] += contrib


def _tc_pool(a0, a1, hp, dv, root, s, bb, bt):
    return pl.pallas_call(
        _pool_body,
        grid=(NP // _BLK,),
        in_specs=[
            pl.BlockSpec((_BLK, D), lambda i: (i, 0)),
            pl.BlockSpec((_BLK, D), lambda i: (i, 0)),
            pl.BlockSpec((_BLK, D), lambda i: (i, 0)),
            pl.BlockSpec((_BLK, 1), lambda i: (i, 0)),
            pl.BlockSpec((1, D), lambda i: (0, 0)),
            pl.BlockSpec((1, D), lambda i: (0, 0)),
            pl.BlockSpec((1, D), lambda i: (0, 0)),
            pl.BlockSpec((_BLK, 1), lambda i: (i, 0)),
        ],
        out_specs=pl.BlockSpec((G, D), lambda i: (0, 0)),
        out_shape=jax.ShapeDtypeStruct((G, D), jnp.float32),
    )(a0, a1, hp, dv, root, s, bb, bt)


def kernel(x, edge_index, edge_attr, batch, atom_emb, lin_W, lin_b, root_emb,
           bond_e0, bond_e1, bond_e2, bn_w, bn_b):
    row2 = jnp.pad(edge_index[0], (0, EP - E), constant_values=NP - 1)
    col2 = jnp.pad(edge_index[1], (0, EP - E), constant_values=NP - 1)
    eaT = edge_attr.T
    ea0 = jnp.pad(eaT[0], (0, EP - E))
    ea1 = jnp.pad(eaT[1], (0, EP - E))
    ea2 = jnp.pad(eaT[2], (0, EP - E))
    xT = jnp.pad(x.T, ((0, 0), (0, NP - N))).reshape(9 * NP)
    aef = jnp.pad(atom_emb.reshape(9 * 119, D), ((0, 1072 - 9 * 119), (0, 0)))
    ct = (bond_e0[:, :, None, None, :] + bond_e1[:, None, :, None, :]
          + bond_e2[:, None, None, :, :]).reshape(L, 60, D)
    ct = jnp.pad(ct, ((0, 0), (0, 4), (0, 0)))
    s_bn = (bn_w / jnp.sqrt(1.0 + 1e-5)).reshape(L, 1, D)
    b_bn = bn_b.reshape(L, 1, D)
    root = root_emb.reshape(L, 1, D)
    wT = jnp.transpose(lin_W, (0, 2, 1))
    wb = lin_b.reshape(L, 1, D)
    bt = jnp.pad(batch, (0, NP - N), constant_values=G).reshape(NP, 1)

    hist2 = _hist_kernel(row2).reshape(NC, NP, 1)
    dv = _tc_dis(hist2[0], hist2[1])
    dis = dv.reshape(NP)
    norm2, idx3 = _norm_kernel(row2, col2, ea0, ea1, ea2, dis)
    h0 = _atom_kernel(xT, aef)

    hp = _tc_matmul(h0, wT[0], wb[0])
    for l in range(L):
        agg2 = _edge_kernel(hp, idx3, norm2, ct[l])
        if l < L - 1:
            hp = _tc_ewmm(agg2[0], agg2[1], hp, dv, root[l], s_bn[l],
                          b_bn[l], wT[l + 1], wb[l + 1])
        else:
            z = _tc_pool(agg2[0], agg2[1], hp, dv, root[l], s_bn[l],
                         b_bn[l], bt)
    return z


# speedup vs baseline: 6.4171x; 2.5403x over previous
import functools

import jax
import jax.numpy as jnp
from jax import lax
from jax.experimental import pallas as pl
from jax.experimental.pallas import tpu as pltpu
from jax.experimental.pallas import tpu_sc as plsc

N = 10000
E = 320000
D = 128
L = 3
G = 64

NC = 2
NS = 16
NW = NC * NS

NP = 10240
EP = ((E + NW * 128 - 1) // (NW * 128)) * (NW * 128)
EP = 327680
ER = EP // 128
CPW = ER // NW
HIST_PT = ER // NS
NPT = NP // NS
NPW = NP // NW

_MESH = plsc.VectorSubcoreMesh(core_axis_name="c", subcore_axis_name="s",
                               num_cores=NC, num_subcores=NS)


@functools.partial(
    pl.kernel,
    out_type=jax.ShapeDtypeStruct((NC * NP,), jnp.float32),
    mesh=_MESH,
    scratch_types=[
        pltpu.VMEM_SHARED((NP, 16), jnp.float32),
        pltpu.VMEM((128, 16), jnp.float32),
        pltpu.VMEM((128,), jnp.int32),
        pltpu.VMEM((NPT, 16), jnp.float32),
        pltpu.VMEM((NPT,), jnp.float32),
    ],
    compiler_params=pltpu.CompilerParams(needs_layout_passes=False),
)
def _hist_kernel(row2, hist_out, hist_sh, ones_v, ridx, dbuf, disv):
    core = lax.axis_index("c")
    sid = lax.axis_index("s")

    one16 = jnp.full((16,), 1.0, jnp.float32)
    zero16 = jnp.zeros((16,), jnp.float32)
    for j in range(128):
        ones_v[j, :] = one16
    for j in range(NPT):
        dbuf[j, :] = zero16
    pltpu.sync_copy(dbuf, hist_sh.at[pl.ds(sid * NPT, NPT)])
    plsc.subcore_barrier()

    def _hist(c, _):
        rr = ((core * NS + sid) * CPW + c) * 128
        pltpu.sync_copy(row2.at[pl.ds(rr, 128)], ridx)
        pltpu.sync_copy(ones_v, hist_sh.at[ridx], add=True)
        return 0
    lax.fori_loop(0, CPW, _hist, 0)
    plsc.subcore_barrier()

    pltpu.sync_copy(hist_sh.at[pl.ds(sid * NPT, NPT)], dbuf)
    zidx = jnp.zeros((16,), jnp.int32)
    for g in range(NPT // 16):
        rowi = jnp.int32(g * 16) + lax.iota(jnp.int32, 16)
        disv[pl.ds(g * 16, 16)] = plsc.load_gather(dbuf, [rowi, zidx])

    pltpu.sync_copy(disv, hist_out.at[pl.ds(core * NP + sid * NPT, NPT)])


@functools.partial(
    pl.kernel,
    out_type=(
        jax.ShapeDtypeStruct((EP,), jnp.float32),
        jax.ShapeDtypeStruct((ER * 8, 128), jnp.int32),
    ),
    mesh=_MESH,
    scratch_types=[
        pltpu.VMEM((8, 128), jnp.int32),
        pltpu.VMEM((128,), jnp.int32),
        pltpu.VMEM((128,), jnp.int32),
        pltpu.VMEM((128,), jnp.int32),
        pltpu.VMEM((128,), jnp.float32),
        pltpu.VMEM((NP,), jnp.float32),
    ],
    compiler_params=pltpu.CompilerParams(needs_layout_passes=False),
)
def _norm_kernel(row2, col2, ea0, ea1, ea2, dis, norm_out, idx3_out,
                 pk, av0, av1, av2, nmv, dis_tab):
    core = lax.axis_index("c")
    sid = lax.axis_index("s")
    w = core * NS + sid

    pltpu.sync_copy(dis, dis_tab)

    def _edge(c, _):
        rr = w * CPW + c
        sl128 = pl.ds(rr * 128, 128)
        pltpu.sync_copy(row2.at[sl128], pk.at[0])
        pltpu.sync_copy(col2.at[sl128], pk.at[1])
        pltpu.sync_copy(ea0.at[sl128], av0)
        pltpu.sync_copy(ea1.at[sl128], av1)
        pltpu.sync_copy(ea2.at[sl128], av2)
        for g in range(8):
            sl = pl.ds(g * 16, 16)
            dr = plsc.load_gather(dis_tab, [pk[0, sl]])
            dc = plsc.load_gather(dis_tab, [pk[1, sl]])
            nmv[sl] = dr * dc
            pk[2, sl] = av0[sl] * 12 + av1[sl] * 2 + av2[sl]
        pltpu.sync_copy(nmv, norm_out.at[sl128])
        pltpu.sync_copy(pk, idx3_out.at[pl.ds(rr * 8, 8)])
        return 0
    lax.fori_loop(0, CPW, _edge, 0)


@functools.partial(
    pl.kernel,
    out_type=jax.ShapeDtypeStruct((NP, D), jnp.float32),
    mesh=_MESH,
    scratch_types=[
        pltpu.VMEM((64,), jnp.int32),
        pltpu.VMEM((64,), jnp.int32),
        pltpu.VMEM((64, D), jnp.float32),
        pltpu.VMEM((64, D), jnp.float32),
        pltpu.SemaphoreType.DMA,
    ],
    compiler_params=pltpu.CompilerParams(needs_layout_passes=False),
)
def _atom_kernel(xT, aef, h0, xv, idxv, acc, gbuf, sem):
    core = lax.axis_index("c")
    sid = lax.axis_index("s")
    w = core * NS + sid

    def _chunk(c, _):
        nb = w * NPW + c * 64
        for i in range(9):
            pltpu.sync_copy(xT.at[pl.ds(i * NP + nb, 64)], xv)
            for g in range(4):
                sl = pl.ds(g * 16, 16)
                idxv[sl] = xv[sl] + jnp.int32(119 * i)
            if i == 0:
                pltpu.async_copy(aef.at[idxv], acc, sem).wait()
            else:
                pltpu.async_copy(aef.at[idxv], gbuf, sem).wait()

                def _accum(j, _):
                    for g2 in range(8):
                        sl2 = pl.ds(g2 * 16, 16)
                        acc[j, sl2] = acc[j, sl2] + gbuf[j, sl2]
                    return 0
                lax.fori_loop(0, 64, _accum, 0)
        pltpu.sync_copy(acc, h0.at[pl.ds(nb, 64)])
        return 0
    lax.fori_loop(0, NPW // 64, _chunk, 0)


@functools.partial(
    pl.kernel,
    out_type=jax.ShapeDtypeStruct((NC, NP, D), jnp.float32),
    mesh=_MESH,
    scratch_types=[
        pltpu.VMEM_SHARED((NP, D), jnp.float32),
        pltpu.VMEM((64, D), jnp.float32),
        pltpu.VMEM((8, 128), jnp.int32),
        pltpu.VMEM((8, 128), jnp.int32),
        pltpu.VMEM((128,), jnp.float32),
        pltpu.VMEM((128,), jnp.float32),
        pltpu.VMEM((128, D), jnp.float32),
        pltpu.VMEM((128, D), jnp.float32),
        pltpu.SemaphoreType.DMA,
        pltpu.SemaphoreType.DMA,
    ],
    compiler_params=pltpu.CompilerParams(needs_layout_passes=False),
)
def _edge_kernel(h, idx3, norm2, ct, agg2,
                 agg_sh, ct_v, pk0, pk1, nm0, nm1, hrow0, hrow1, sem0, sem1):
    core = lax.axis_index("c")
    sid = lax.axis_index("s")
    w = core * NS + sid

    zero16 = jnp.zeros((16,), jnp.float32)
    for j in range(128):
        for g in range(8):
            hrow0[j, pl.ds(g * 16, 16)] = zero16
    for t in range(NPT // 128):
        pltpu.sync_copy(hrow0, agg_sh.at[pl.ds(sid * NPT + t * 128, 128)])
    pltpu.sync_copy(ct, ct_v)
    plsc.subcore_barrier()

    def _load_idx(c, pk, nm):
        rr = w * CPW + c
        pltpu.sync_copy(idx3.at[pl.ds(rr * 8, 8)], pk)
        pltpu.sync_copy(norm2.at[pl.ds(rr * 128, 128)], nm)

    def _compute(pk, nm, hrow, msg):
        csets = [jnp.int32(g * 16) + lax.iota(jnp.int32, 16)
                 for g in range(8)]
        two16 = jnp.full((16,), 2, dtype=jnp.int32)

        @plsc.parallel_loop(0, 128, unroll=2)
        def _edge(e, pk=pk, nm=nm, hrow=hrow):
            ef = jnp.full((16,), e, dtype=jnp.int32)
            cef = plsc.load_gather(pk, [two16, ef])
            nef = plsc.load_gather(nm, [ef])
            for g in range(8):
                cv = csets[g]
                hv = plsc.load_gather(hrow, [ef, cv])
                ev = plsc.load_gather(ct_v, [cef, cv])
                m = jnp.maximum(hv + ev, 0.0) * nef
                plsc.store_scatter(hrow, [ef, cv], m)
        pltpu.sync_copy(hrow, agg_sh.at[pk.at[1]], add=True)

    _load_idx(0, pk0, nm0)
    pltpu.async_copy(h.at[pk0.at[0]], hrow0, sem0)
    _load_idx(1, pk1, nm1)
    pltpu.async_copy(h.at[pk1.at[0]], hrow1, sem1)

    bufs = ((pk0, nm0, hrow0, sem0), (pk1, nm1, hrow1, sem1))

    def _pair(it, _):
        for b in range(2):
            pk, nm, hrow, sem = bufs[b]
            c = it * 2 + b
            pltpu.make_async_copy(h.at[pk.at[0]], hrow, sem).wait()
            _compute(pk, nm, hrow, hrow)

            @pl.when(c + 2 < CPW)
            def _(pk=pk, nm=nm, hrow=hrow, sem=sem, c=c):
                _load_idx(c + 2, pk, nm)
                pltpu.async_copy(h.at[pk.at[0]], hrow, sem)
        return 0
    lax.fori_loop(0, CPW // 2, _pair, 0)
    plsc.subcore_barrier()

    pltpu.sync_copy(agg_sh.at[pl.ds(sid * NPT, NPT)],
                    agg2.at[core].at[pl.ds(sid * NPT, NPT)])


_BLK = 1024


def _dis_body(h0_ref, h1_ref, o_ref):
    o_ref[...] = lax.rsqrt(h0_ref[...] + h1_ref[...] + 1.0)


def _tc_dis(hist0, hist1):
    return pl.pallas_call(
        _dis_body,
        out_shape=jax.ShapeDtypeStruct((NP, 1), jnp.float32),
    )(hist0, hist1)


def _mm_body(x_ref, w_ref, b_ref, o_ref):
    o_ref[...] = jnp.dot(x_ref[...], w_ref[...],
                         preferred_element_type=jnp.float32) + b_ref[...]


def _tc_matmul(hx, wT, b):
    return pl.pallas_call(
        _mm_body,
        grid=(NP // _BLK,),
        in_specs=[
            pl.BlockSpec((_BLK, D), lambda i: (i, 0)),
            pl.BlockSpec((D, D), lambda i: (0, 0)),
            pl.BlockSpec((1, D), lambda i: (0, 0)),
        ],
        out_specs=pl.BlockSpec((_BLK, D), lambda i: (i, 0)),
        out_shape=jax.ShapeDtypeStruct((NP, D), jnp.float32),
    )(hx, wT, b)


def _ewmm_body(a0_ref, a1_ref, hp_ref, dv_ref, root_ref, s_ref, bb_ref,
               w_ref, wb_ref, o_ref):
    dd = dv_ref[...]
    hm = (a0_ref[...] + a1_ref[...]
          + jnp.maximum(hp_ref[...] + root_ref[...], 0.0) * (dd * dd))
    hm = hm * s_ref[...] + bb_ref[...]
    hm = jnp.maximum(hm, 0.0)
    o_ref[...] = jnp.dot(hm, w_ref[...],
                         preferred_element_type=jnp.float32) + wb_ref[...]


def _tc_ewmm(a0, a1, hp, dv, root, s, bb, wT, wb):
    return pl.pallas_call(
        _ewmm_body,
        grid=(NP // _BLK,),
        in_specs=[
            pl.BlockSpec((_BLK, D), lambda i: (i, 0)),
            pl.BlockSpec((_BLK, D), lambda i: (i, 0)),
            pl.BlockSpec((_BLK, D), lambda i: (i, 0)),
            pl.BlockSpec((_BLK, 1), lambda i: (i, 0)),
            pl.BlockSpec((1, D), lambda i: (0, 0)),
            pl.BlockSpec((1, D), lambda i: (0, 0)),
            pl.BlockSpec((1, D), lambda i: (0, 0)),
            pl.BlockSpec((D, D), lambda i: (0, 0)),
            pl.BlockSpec((1, D), lambda i: (0, 0)),
        ],
        out_specs=pl.BlockSpec((_BLK, D), lambda i: (i, 0)),
        out_shape=jax.ShapeDtypeStruct((NP, D), jnp.float32),
    )(a0, a1, hp, dv, root, s, bb, wT, wb)


def _pool_body(a0_ref, a1_ref, hp_ref, dv_ref, root_ref, s_ref, bb_ref,
               bt_ref, o_ref):
    dd = dv_ref[...]
    hm = (a0_ref[...] + a1_ref[...]
          + jnp.maximum(hp_ref[...] + root_ref[...], 0.0) * (dd * dd))
    hm = hm * s_ref[...] + bb_ref[...]
    gids = lax.broadcasted_iota(jnp.int32, (G, _BLK), 0)
    onehot = (gids == bt_ref[...].reshape(1, _BLK)).astype(jnp.float32)
    contrib = jnp.dot(onehot, hm, preferred_element_type=jnp.float32)

    @pl.when(pl.program_id(0) == 0)
    def _():
        o_ref[...] = jnp.zeros_like(o_ref)

    o_ref[...] += contrib


def _tc_pool(a0, a1, hp, dv, root, s, bb, bt):
    return pl.pallas_call(
        _pool_body,
        grid=(NP // _BLK,),
        in_specs=[
            pl.BlockSpec((_BLK, D), lambda i: (i, 0)),
            pl.BlockSpec((_BLK, D), lambda i: (i, 0)),
            pl.BlockSpec((_BLK, D), lambda i: (i, 0)),
            pl.BlockSpec((_BLK, 1), lambda i: (i, 0)),
            pl.BlockSpec((1, D), lambda i: (0, 0)),
            pl.BlockSpec((1, D), lambda i: (0, 0)),
            pl.BlockSpec((1, D), lambda i: (0, 0)),
            pl.BlockSpec((_BLK, 1), lambda i: (i, 0)),
        ],
        out_specs=pl.BlockSpec((G, D), lambda i: (0, 0)),
        out_shape=jax.ShapeDtypeStruct((G, D), jnp.float32),
    )(a0, a1, hp, dv, root, s, bb, bt)


def kernel(x, edge_index, edge_attr, batch, atom_emb, lin_W, lin_b, root_emb,
           bond_e0, bond_e1, bond_e2, bn_w, bn_b):
    row2 = jnp.pad(edge_index[0], (0, EP - E), constant_values=NP - 1)
    col2 = jnp.pad(edge_index[1], (0, EP - E), constant_values=NP - 1)
    eaT = edge_attr.T
    ea0 = jnp.pad(eaT[0], (0, EP - E))
    ea1 = jnp.pad(eaT[1], (0, EP - E))
    ea2 = jnp.pad(eaT[2], (0, EP - E))
    xT = jnp.pad(x.T, ((0, 0), (0, NP - N))).reshape(9 * NP)
    aef = jnp.pad(atom_emb.reshape(9 * 119, D), ((0, 1072 - 9 * 119), (0, 0)))
    ct = (bond_e0[:, :, None, None, :] + bond_e1[:, None, :, None, :]
          + bond_e2[:, None, None, :, :]).reshape(L, 60, D)
    ct = jnp.pad(ct, ((0, 0), (0, 4), (0, 0)))
    s_bn = (bn_w / jnp.sqrt(1.0 + 1e-5)).reshape(L, 1, D)
    b_bn = bn_b.reshape(L, 1, D)
    root = root_emb.reshape(L, 1, D)
    wT = jnp.transpose(lin_W, (0, 2, 1))
    wb = lin_b.reshape(L, 1, D)
    bt = jnp.pad(batch, (0, NP - N), constant_values=G).reshape(NP, 1)

    hist2 = _hist_kernel(row2).reshape(NC, NP, 1)
    dv = _tc_dis(hist2[0], hist2[1])
    dis = dv.reshape(NP)
    norm2, idx3 = _norm_kernel(row2, col2, ea0, ea1, ea2, dis)
    h0 = _atom_kernel(xT, aef)

    hp = _tc_matmul(h0, wT[0], wb[0])
    for l in range(L):
        agg2 = _edge_kernel(hp, idx3, norm2, ct[l])
        if l < L - 1:
            hp = _tc_ewmm(agg2[0], agg2[1], hp, dv, root[l], s_bn[l],
                          b_bn[l], wT[l + 1], wb[l + 1])
        else:
            z = _tc_pool(agg2[0], agg2[1], hp, dv, root[l], s_bn[l],
                         b_bn[l], bt)
    return z
